# baseline (speedup 1.0000x reference)
import functools
import jax
import jax.numpy as jnp
from jax import lax
from jax.experimental import pallas as pl
from jax.experimental.pallas import tpu as pltpu
from jax.experimental.pallas import tpu_sc as plsc

_B, _H, _V, _D = 8, 16, 8192, 64
_N = 4096
_NC, _NS = 2, 16
_NW = _NC * _NS
_PAIRS = _B * _H
_PPW = _PAIRS // _NW
_CH = 128
_NCH = _N // _CH
_BI = 2 * _CH
_BPP = _N // _BI
_NB = _PPW * _BPP


def _sc_gather(t2, k2, o2):
    mesh = plsc.VectorSubcoreMesh(core_axis_name="c", subcore_axis_name="s")

    @functools.partial(
        pl.kernel,
        out_type=jax.ShapeDtypeStruct((_PAIRS, _N // 2, 2 * _D), jnp.float32),
        mesh=mesh,
        scratch_types=[
            pltpu.VMEM((_NCH, _CH), jnp.int32),
            pltpu.VMEM((_NCH, _CH), jnp.int32),
            pltpu.VMEM((_BI, 2 * _D), jnp.float32),
            pltpu.VMEM((_BI, 2 * _D), jnp.float32),
            pltpu.VMEM((_BI // 2, 2 * _D), jnp.float32),
            pltpu.VMEM((_BI // 2, 2 * _D), jnp.float32),
            pltpu.SemaphoreType.DMA,
            pltpu.SemaphoreType.DMA,
            pltpu.SemaphoreType.DMA,
            pltpu.SemaphoreType.DMA,
        ],
    )
    def body(t_hbm, k_hbm, o_hbm, out_hbm,
             kv, ov, pad0, pad1, cmp0, cmp1, gs0, gs1, ss0, ss1):
        cid = lax.axis_index("c")
        sid = lax.axis_index("s")
        wid = sid * _NC + cid
        pltpu.sync_copy(k_hbm, kv)
        pltpu.sync_copy(o_hbm, ov)

        def issue_gathers(b, pad, sem):
            q = b // _BPP
            r = b % _BPP
            p = wid * _PPW + q
            for j in range(2):
                pltpu.async_copy(
                    t_hbm.at[p].at[kv.at[2 * r + j]],
                    pad.at[pl.ds(j * _CH, _CH)],
                    sem,
                )

        def wait_gathers(pad, sem):
            pltpu.make_async_copy(
                out_hbm.at[0].at[pl.ds(0, _BI)], pad, sem
            ).wait()

        def compact(b, pad, cmp):
            r = b % _BPP

            def row16(g, carry):
                cc = 2 * r + g // (_CH // 16)
                gi = g % (_CH // 16)
                hv = ov[cc, pl.ds(gi * 16, 16)]
                for l in range(16):
                    i = g * 16 + l
                    hoff = hv[l]
                    half = (l & 1) * _D
                    for j in range(_D // 16):
                        cmp[i // 2, pl.ds(half + j * 16, 16)] = (
                            pad[i, pl.ds(hoff + j * 16, 16)]
                        )
                return carry

            lax.fori_loop(0, _BI // 16, row16, 0)

        def issue_store(b, cmp, sem):
            q = b // _BPP
            r = b % _BPP
            p = wid * _PPW + q
            pltpu.async_copy(
                cmp, out_hbm.at[p].at[pl.ds(r * (_BI // 2), _BI // 2)], sem
            )

        def wait_store(cmp, sem):
            pltpu.make_async_copy(
                cmp, out_hbm.at[0].at[pl.ds(0, _BI // 2)], sem
            ).wait()

        issue_gathers(0, pad0, gs0)

        def loop(i, carry):
            b0 = 2 * i
            b1 = 2 * i + 1
            issue_gathers(b1, pad1, gs1)
            wait_gathers(pad0, gs0)

            @pl.when(i > 0)
            def _():
                wait_store(cmp0, ss0)

            compact(b0, pad0, cmp0)
            issue_store(b0, cmp0, ss0)

            @pl.when(i < _NB // 2 - 1)
            def _():
                issue_gathers(b0 + 2, pad0, gs0)

            wait_gathers(pad1, gs1)

            @pl.when(i > 0)
            def _():
                wait_store(cmp1, ss1)

            compact(b1, pad1, cmp1)
            issue_store(b1, cmp1, ss1)
            return carry

        lax.fori_loop(0, _NB // 2, loop, 0)
        wait_store(cmp0, ss0)
        wait_store(cmp1, ss1)

    return body(t2, k2, o2)


def kernel(t, idx):
    t2 = t.reshape(_PAIRS, _V // 2, 2 * _D)
    idx = idx.astype(jnp.int32)
    k2 = lax.shift_right_logical(idx, 1).reshape(_NCH, _CH)
    o2 = ((idx & 1) * _D).reshape(_NCH, _CH)
    out2 = _sc_gather(t2, k2, o2)
    return out2.reshape(_B, _H, _N, _D)


# speedup vs baseline: 2.6852x; 2.6852x over previous
import functools
import jax
import jax.numpy as jnp
from jax import lax
from jax.experimental import pallas as pl
from jax.experimental.pallas import tpu as pltpu
from jax.experimental.pallas import tpu_sc as plsc

_B, _H, _V, _D = 8, 16, 8192, 64
_N = 4096
_NC, _NS = 2, 16
_NW = _NC * _NS
_R = _B * _H * _D
_RPW = _R // _NW
_RPB = 4
_NB = _RPW // _RPB


def _sc_gather(tt, idx):
    mesh = plsc.VectorSubcoreMesh(core_axis_name="c", subcore_axis_name="s")

    @functools.partial(
        pl.kernel,
        out_type=jax.ShapeDtypeStruct((_R, _N), jnp.float32),
        mesh=mesh,
        compiler_params=pltpu.CompilerParams(needs_layout_passes=False),
        scratch_types=[
            pltpu.VMEM((_N,), jnp.int32),
            pltpu.VMEM((_RPB, _V), jnp.float32),
            pltpu.VMEM((_RPB, _V), jnp.float32),
            pltpu.VMEM((_RPB, _N), jnp.float32),
            pltpu.VMEM((_RPB, _N), jnp.float32),
            pltpu.SemaphoreType.DMA,
            pltpu.SemaphoreType.DMA,
            pltpu.SemaphoreType.DMA,
            pltpu.SemaphoreType.DMA,
        ],
    )
    def body(t_hbm, idx_hbm, out_hbm,
             idx_v, rb0, rb1, cb0, cb1, gs0, gs1, ss0, ss1):
        cid = lax.axis_index("c")
        sid = lax.axis_index("s")
        wid = sid * _NC + cid
        base = wid * _RPW
        pltpu.sync_copy(idx_hbm, idx_v)

        def issue_read(b, rb, sem):
            pltpu.async_copy(
                t_hbm.at[pl.ds(base + b * _RPB, _RPB)], rb, sem
            )

        def wait_read(rb, sem):
            pltpu.make_async_copy(
                t_hbm.at[pl.ds(0, _RPB)], rb, sem
            ).wait()

        def gather(rb, cb):
            def grp(g, carry):
                iv = idx_v[pl.ds(g * 16, 16)]
                for k in range(_RPB):
                    kv = jnp.full((16,), k, jnp.int32)
                    cb[k, pl.ds(g * 16, 16)] = plsc.load_gather(
                        rb, [kv, iv]
                    )
                return carry

            lax.fori_loop(0, _N // 16, grp, 0)

        def issue_store(b, cb, sem):
            pltpu.async_copy(
                cb, out_hbm.at[pl.ds(base + b * _RPB, _RPB)], sem
            )

        def wait_store(cb, sem):
            pltpu.make_async_copy(
                cb, out_hbm.at[pl.ds(0, _RPB)], sem
            ).wait()

        issue_read(0, rb0, gs0)

        def loop(i, carry):
            b0 = 2 * i
            b1 = 2 * i + 1
            issue_read(b1, rb1, gs1)
            wait_read(rb0, gs0)

            @pl.when(i > 0)
            def _():
                wait_store(cb0, ss0)

            gather(rb0, cb0)
            issue_store(b0, cb0, ss0)

            @pl.when(i < _NB // 2 - 1)
            def _():
                issue_read(b0 + 2, rb0, gs0)

            wait_read(rb1, gs1)

            @pl.when(i > 0)
            def _():
                wait_store(cb1, ss1)

            gather(rb1, cb1)
            issue_store(b1, cb1, ss1)
            return carry

        lax.fori_loop(0, _NB // 2, loop, 0)
        wait_store(cb0, ss0)
        wait_store(cb1, ss1)

    return body(tt, idx)


def kernel(t, idx):
    tt = jnp.transpose(t, (0, 1, 3, 2)).reshape(_R, _V)
    out_t = _sc_gather(tt, idx.astype(jnp.int32))
    return jnp.transpose(out_t.reshape(_B, _H, _D, _N), (0, 1, 3, 2))


# speedup vs baseline: 7.3814x; 2.7489x over previous
import functools
import jax
import jax.numpy as jnp
from jax import lax
from jax.experimental import pallas as pl
from jax.experimental.pallas import tpu as pltpu
from jax.experimental.pallas import tpu_sc as plsc

_B, _H, _V, _D = 8, 16, 8192, 64
_N = 4096
_NC, _NS = 2, 16
_NW = _NC * _NS
_R = _B * _H * _D
_RPW = _R // _NW
_RPB = 4
_NB = _RPW // _RPB


def _sc_gather(tt, idx):
    mesh = plsc.VectorSubcoreMesh(core_axis_name="c", subcore_axis_name="s")

    @functools.partial(
        pl.kernel,
        out_type=jax.ShapeDtypeStruct((_R, _N), jnp.float32),
        mesh=mesh,
        compiler_params=pltpu.CompilerParams(needs_layout_passes=False),
        scratch_types=[
            pltpu.VMEM((_N,), jnp.int32),
            pltpu.VMEM((_RPB, _V), jnp.float32),
            pltpu.VMEM((_RPB, _V), jnp.float32),
            pltpu.VMEM((_RPB, _N), jnp.float32),
            pltpu.VMEM((_RPB, _N), jnp.float32),
            pltpu.SemaphoreType.DMA,
            pltpu.SemaphoreType.DMA,
            pltpu.SemaphoreType.DMA,
            pltpu.SemaphoreType.DMA,
        ],
    )
    def body(t_hbm, idx_hbm, out_hbm,
             idx_v, rb0, rb1, cb0, cb1, gs0, gs1, ss0, ss1):
        cid = lax.axis_index("c")
        sid = lax.axis_index("s")
        wid = sid * _NC + cid
        base = wid * _RPW
        pltpu.sync_copy(idx_hbm, idx_v)

        def issue_read(b, rb, sem):
            pltpu.async_copy(
                t_hbm.at[pl.ds(base + b * _RPB, _RPB)], rb, sem
            )

        def wait_read(rb, sem):
            pltpu.make_async_copy(
                t_hbm.at[pl.ds(0, _RPB)], rb, sem
            ).wait()

        kvs = [jnp.full((16,), k, jnp.int32) for k in range(_RPB)]

        def gather(rb, cb):
            @plsc.parallel_loop(0, _N // 16, 1, unroll=8)
            def grp(g):
                iv = idx_v[pl.ds(g * 16, 16)]
                for k in range(_RPB):
                    cb[k, pl.ds(g * 16, 16)] = plsc.load_gather(
                        rb, [kvs[k], iv]
                    )

        def issue_store(b, cb, sem):
            pltpu.async_copy(
                cb, out_hbm.at[pl.ds(base + b * _RPB, _RPB)], sem
            )

        def wait_store(cb, sem):
            pltpu.make_async_copy(
                cb, out_hbm.at[pl.ds(0, _RPB)], sem
            ).wait()

        issue_read(0, rb0, gs0)

        def loop(i, carry):
            b0 = 2 * i
            b1 = 2 * i + 1
            issue_read(b1, rb1, gs1)
            wait_read(rb0, gs0)

            @pl.when(i > 0)
            def _():
                wait_store(cb0, ss0)

            gather(rb0, cb0)
            issue_store(b0, cb0, ss0)

            @pl.when(i < _NB // 2 - 1)
            def _():
                issue_read(b0 + 2, rb0, gs0)

            wait_read(rb1, gs1)

            @pl.when(i > 0)
            def _():
                wait_store(cb1, ss1)

            gather(rb1, cb1)
            issue_store(b1, cb1, ss1)
            return carry

        lax.fori_loop(0, _NB // 2, loop, 0)
        wait_store(cb0, ss0)
        wait_store(cb1, ss1)

    return body(tt, idx)


def kernel(t, idx):
    tt = jnp.transpose(t, (0, 1, 3, 2)).reshape(_R, _V)
    out_t = _sc_gather(tt, idx.astype(jnp.int32))
    return jnp.transpose(out_t.reshape(_B, _H, _D, _N), (0, 1, 3, 2))
